# Initial kernel scaffold; baseline (speedup 1.0000x reference)
import jax
import jax.numpy as jnp
from jax.experimental import pallas as pl


def kernel(x, edge_index, tar_ei, xcn_W1, xcn_b1, xcn_g, xcn_be, xcn_W2, xcn_b2, xij_W1, xij_b1, xij_g, xij_be, xij_W2, xij_b2, lin_W1, lin_b1, lin_g, lin_be, lin_W2, lin_b2):
    raise NotImplementedError("write your pallas kernel here")


# speedup vs baseline: 1.5234x; 1.5234x over previous
import functools

import jax
import jax.numpy as jnp
from jax import lax
from jax.experimental import pallas as pl
from jax.experimental.pallas import tpu as pltpu
from jax.experimental.pallas import tpu_sc as plsc

N = 10000
E = 320000
B = 4096
D = 128
H = 128
OUT = 1
BETA = 1.0
EPS = 1e-5

L = 16
NC = 2
NS = 16
NW = NC * NS
W = 384
NPAD = 32 * W
RPT = 313
NROWS = NW * RPT
ECHUNK = 4000
BPW = B // NW
BB = 256

_GDN = lax.GatherDimensionNumbers(
    offset_dims=(), collapsed_slice_dims=(0,), start_index_map=(0,))


def _rot(a, idx):
    return lax.gather(a, idx[:, None], _GDN, (1,),
                      mode=lax.GatherScatterMode.PROMISE_IN_BOUNDS)


def _sc_build(esrc_hbm, edst_hbm, abits_hbm, bits_v, sbuf_v, dbuf_v):
    wid = lax.axis_index("s") * NC + lax.axis_index("c")
    r0 = wid * RPT

    lanes = lax.iota(jnp.int32, L)
    lane0 = lanes == 0
    lanebit = jnp.left_shift(jnp.ones((L,), jnp.int32), lanes)
    rotidx = [(lanes + k) & (L - 1) for k in (1, 2, 4, 8)]
    zero16 = jnp.zeros((L,), jnp.int32)

    def zero_body(i, carry):
        bits_v[pl.ds(i * L, L)] = zero16
        return carry

    lax.fori_loop(0, (RPT * W) // L, zero_body, 0)

    def chunk_body(c, carry):
        pltpu.sync_copy(esrc_hbm.at[pl.ds(c * ECHUNK, ECHUNK)], sbuf_v)
        pltpu.sync_copy(edst_hbm.at[pl.ds(c * ECHUNK, ECHUNK)], dbuf_v)

        def vec_body(v, inner):
            s16 = sbuf_v[pl.ds(v * L, L)]
            rloc = s16 - r0
            lb = jnp.where((rloc >= 0) & (rloc < RPT), lanebit, 0)
            for ix in rotidx:
                lb = lb | _rot(lb, ix)
            packed = lb[0]

            @pl.when(packed != 0)
            def _():
                d16 = dbuf_v[pl.ds(v * L, L)]
                wv = rloc * W + jnp.right_shift(d16, 5)
                bv = jnp.left_shift(jnp.ones_like(d16), d16 & 31)
                for l in range(L):
                    @pl.when((packed >> l) & 1 != 0)
                    def _():
                        wl = wv[l]
                        bl = bv[l]
                        old = bits_v[pl.ds(wl, L)]
                        bits_v[pl.ds(wl, L)] = jnp.where(lane0, old | bl, old)

            return inner

        lax.fori_loop(0, ECHUNK // L, vec_body, 0)
        return carry

    lax.fori_loop(0, E // ECHUNK, chunk_body, 0)
    pltpu.sync_copy(bits_v, abits_hbm.at[pl.ds(r0 * W, RPT * W)])


def _sc_gather(abits_hbm, x_hbm, tari_hbm, tarj_hbm,
               rowsi_hbm, rowsj_hbm, xi_hbm, xj_hbm,
               idx_v, rows_v, xrow_v, sem):
    wid = lax.axis_index("s") * NC + lax.axis_index("c")
    base = wid * BPW

    pltpu.sync_copy(tari_hbm.at[pl.ds(base, BPW)], idx_v)
    pltpu.async_copy(abits_hbm.at[idx_v], rows_v, sem).wait()
    pltpu.sync_copy(rows_v, rowsi_hbm.at[pl.ds(base, BPW)])
    pltpu.async_copy(x_hbm.at[idx_v], xrow_v, sem).wait()
    pltpu.sync_copy(xrow_v, xi_hbm.at[pl.ds(base, BPW)])

    pltpu.sync_copy(tarj_hbm.at[pl.ds(base, BPW)], idx_v)
    pltpu.async_copy(abits_hbm.at[idx_v], rows_v, sem).wait()
    pltpu.sync_copy(rows_v, rowsj_hbm.at[pl.ds(base, BPW)])
    pltpu.async_copy(x_hbm.at[idx_v], xrow_v, sem).wait()
    pltpu.sync_copy(xrow_v, xj_hbm.at[pl.ds(base, BPW)])


@functools.lru_cache(maxsize=None)
def _sc_kernels():
    mesh = plsc.VectorSubcoreMesh(core_axis_name="c", subcore_axis_name="s",
                                  num_cores=NC, num_subcores=NS)
    build = pl.kernel(
        _sc_build,
        out_type=jax.ShapeDtypeStruct((NROWS * W,), jnp.int32),
        mesh=mesh,
        scratch_types=[
            pltpu.VMEM((RPT * W,), jnp.int32),
            pltpu.VMEM((ECHUNK,), jnp.int32),
            pltpu.VMEM((ECHUNK,), jnp.int32),
        ],
    )
    gather = pl.kernel(
        _sc_gather,
        out_type=(
            jax.ShapeDtypeStruct((B, W), jnp.int32),
            jax.ShapeDtypeStruct((B, W), jnp.int32),
            jax.ShapeDtypeStruct((B, D), jnp.float32),
            jax.ShapeDtypeStruct((B, D), jnp.float32),
        ),
        mesh=mesh,
        scratch_types=[
            pltpu.VMEM((BPW,), jnp.int32),
            pltpu.VMEM((BPW, W), jnp.int32),
            pltpu.VMEM((BPW, D), jnp.float32),
            pltpu.SemaphoreType.DMA,
        ],
    )
    return build, gather


def _mlp(h, W1, b1, g, be, W2, b2):
    h = jnp.dot(h, W1, preferred_element_type=jnp.float32) + b1
    mu = jnp.mean(h, axis=-1, keepdims=True)
    var = jnp.mean((h - mu) ** 2, axis=-1, keepdims=True)
    h = (h - mu) * lax.rsqrt(var + EPS) * g + be
    h = jnp.maximum(h, 0.0)
    return jnp.dot(h, W2, preferred_element_type=jnp.float32) + b2


def _tc_body(rowsi_ref, rowsj_ref, xi_ref, xj_ref, xperm_ref,
             xcn_W1, xcn_b1, xcn_g, xcn_be, xcn_W2, xcn_b2,
             xij_W1, xij_b1, xij_g, xij_be, xij_W2, xij_b2,
             lin_W1, lin_b1, lin_g, lin_be, lin_W2, lin_b2,
             out_ref):
    cnw = rowsi_ref[...] & rowsj_ref[...]
    acc = jnp.zeros((BB, D), jnp.float32)
    for p in range(32):
        mask_p = jnp.int32(-(2 ** 31)) if p == 31 else jnp.int32(1 << p)
        bitf = (cnw & mask_p).astype(jnp.float32)
        acc = acc + jnp.dot(bitf, xperm_ref[p],
                            preferred_element_type=jnp.float32)
    xij = _mlp(xi_ref[...] * xj_ref[...],
               xij_W1[...], xij_b1[...], xij_g[...], xij_be[...],
               xij_W2[...], xij_b2[...])
    xcnh = _mlp(acc,
                xcn_W1[...], xcn_b1[...], xcn_g[...], xcn_be[...],
                xcn_W2[...], xcn_b2[...])
    out_ref[...] = _mlp(xcnh * BETA + xij,
                        lin_W1[...], lin_b1[...], lin_g[...], lin_be[...],
                        lin_W2[...], lin_b2[...])


def _tc_score(rowsi, rowsj, xi, xj, xperm, wts):
    full = lambda shape: pl.BlockSpec(shape, lambda i: (0,) * len(shape))
    in_specs = [
        pl.BlockSpec((BB, W), lambda i: (i, 0)),
        pl.BlockSpec((BB, W), lambda i: (i, 0)),
        pl.BlockSpec((BB, D), lambda i: (i, 0)),
        pl.BlockSpec((BB, D), lambda i: (i, 0)),
        full((32, W, D)),
    ] + [full(w.shape) for w in wts]
    return pl.pallas_call(
        _tc_body,
        grid=(B // BB,),
        in_specs=in_specs,
        out_specs=pl.BlockSpec((BB, OUT), lambda i: (i, 0)),
        out_shape=jax.ShapeDtypeStruct((B, OUT), jnp.float32),
        compiler_params=pltpu.CompilerParams(
            dimension_semantics=("arbitrary",)),
    )(rowsi, rowsj, xi, xj, xperm, *wts)


def kernel(x, edge_index, tar_ei,
           xcn_W1, xcn_b1, xcn_g, xcn_be, xcn_W2, xcn_b2,
           xij_W1, xij_b1, xij_g, xij_be, xij_W2, xij_b2,
           lin_W1, lin_b1, lin_g, lin_be, lin_W2, lin_b2):
    sc_build, sc_gather = _sc_kernels()
    abits = sc_build(edge_index[0], edge_index[1])
    rowsi, rowsj, xi, xj = sc_gather(
        abits.reshape(NROWS, W), x, tar_ei[0], tar_ei[1])

    xpad = jnp.pad(x, ((0, NPAD - N), (0, 0)))
    scale = jnp.concatenate([2.0 ** -jnp.arange(31, dtype=jnp.float32),
                             jnp.array([-(2.0 ** -31)], jnp.float32)])
    xperm = xpad.reshape(W, 32, D).transpose(1, 0, 2) * scale[:, None, None]

    wts = [xcn_W1, xcn_b1.reshape(1, H), xcn_g.reshape(1, H),
           xcn_be.reshape(1, H), xcn_W2, xcn_b2.reshape(1, H),
           xij_W1, xij_b1.reshape(1, H), xij_g.reshape(1, H),
           xij_be.reshape(1, H), xij_W2, xij_b2.reshape(1, H),
           lin_W1, lin_b1.reshape(1, H), lin_g.reshape(1, H),
           lin_be.reshape(1, H), lin_W2, lin_b2.reshape(1, OUT)]
    return _tc_score(rowsi, rowsj, xi, xj, xperm, wts)


# speedup vs baseline: 5.5807x; 3.6634x over previous
import functools

import jax
import jax.numpy as jnp
from jax import lax
from jax.experimental import pallas as pl
from jax.experimental.pallas import tpu as pltpu
from jax.experimental.pallas import tpu_sc as plsc

N = 10000
E = 320000
B = 4096
D = 128
H = 128
OUT = 1
BETA = 1.0
EPS = 1e-5

L = 16
NC = 2
NS = 16
NW = NC * NS
W = 384
NPAD = 32 * W
RPT = 313
NROWS = NW * RPT
BPW = B // NW
BB = 256
EPW = E // NW
CAP = 512
MAGIC = 13401
SENT = RPT * W * 32
WV = 8

_GDN = lax.GatherDimensionNumbers(
    offset_dims=(), collapsed_slice_dims=(0,), start_index_map=(0,))


def _rot(a, idx):
    return lax.gather(a, idx[:, None], _GDN, (1,),
                      mode=lax.GatherScatterMode.PROMISE_IN_BOUNDS)


def _sc_scan(esrc_hbm, edst_hbm, bkt_hbm, bkt_v, cnt_v, sbuf_v, dbuf_v):
    wid = lax.axis_index("s") * NC + lax.axis_index("c")
    lanes = lax.iota(jnp.int32, L)
    lane0 = lanes == 0
    zero16 = jnp.zeros((L,), jnp.int32)

    def init_body(i, carry):
        o = i // (CAP // L)
        bkt_v[pl.ds(i * L, L)] = zero16 + (o + 1) * SENT
        return carry

    lax.fori_loop(0, (NW * CAP) // L, init_body, 0)
    cnt_v[pl.ds(0, L)] = zero16
    cnt_v[pl.ds(L, L)] = zero16
    cnt_v[pl.ds(2 * L, L)] = zero16

    e0 = wid * EPW
    pltpu.sync_copy(esrc_hbm.at[pl.ds(e0, EPW)], sbuf_v)
    pltpu.sync_copy(edst_hbm.at[pl.ds(e0, EPW)], dbuf_v)

    def scan_body(v, carry):
        s16 = sbuf_v[pl.ds(v * L, L)]
        d16 = dbuf_v[pl.ds(v * L, L)]
        ov = jnp.right_shift(s16 * MAGIC, 22)
        pv = s16 * (W * 32) + d16
        for l in range(L):
            rp = _rot(pv, (lanes + l) & (L - 1))
            o_l = ov[l]
            cw = cnt_v[pl.ds(o_l, L)]
            c = jnp.minimum(cw[0], CAP - L)
            bkt_v[pl.ds(o_l * CAP + c, L)] = rp
            cnt_v[pl.ds(o_l, L)] = jnp.where(lane0, c + 1, cw)
        return carry

    lax.fori_loop(0, EPW // L, scan_body, 0)

    for o in range(NW):
        c = jnp.minimum(cnt_v[pl.ds(o, L)][0], CAP - L)
        bkt_v[pl.ds(o * CAP + c, L)] = zero16 + (o + 1) * SENT

    pltpu.sync_copy(bkt_v, bkt_hbm.at[pl.ds(wid * NW * CAP, NW * CAP)])


def _sc_apply(bkt_hbm, abits_hbm, bits_v, stg_v, sem):
    wid = lax.axis_index("s") * NC + lax.axis_index("c")
    lanes = lax.iota(jnp.int32, L)
    lane0 = lanes == 0
    zero16 = jnp.zeros((L,), jnp.int32)

    def issue(g):
        h = (g % 2) * (WV * CAP)
        return [pltpu.async_copy(
            bkt_hbm.at[pl.ds(((g * WV + k) * NW + wid) * CAP, CAP)],
            stg_v.at[pl.ds(h + k * CAP, CAP)], sem) for k in range(WV)]

    pend = {0: issue(0)}

    def zero_body(i, carry):
        bits_v[pl.ds(i * L, L)] = zero16
        return carry

    lax.fori_loop(0, (RPT * W + L) // L, zero_body, 0)
    pend[1] = issue(1)

    for g in range(NW // WV):
        for cp in pend.pop(g):
            cp.wait()
        h = (g % 2) * (WV * CAP)

        def vec_body(v, carry, h=h):
            pv = stg_v[pl.ds(h + v * L, L)]
            wv = jnp.right_shift(pv, 5) - wid * (RPT * W)
            bv = jnp.left_shift(jnp.ones((L,), jnp.int32), pv & 31)
            for l in range(L):
                wl = wv[l]
                rb = _rot(bv, (lanes + l) & (L - 1))
                old = bits_v[pl.ds(wl, L)]
                bits_v[pl.ds(wl, L)] = jnp.where(lane0, old | rb, old)
            return carry

        lax.fori_loop(0, (WV * CAP) // L, vec_body, 0)
        if g + 2 < NW // WV:
            pend[g + 2] = issue(g + 2)

    pltpu.sync_copy(bits_v.at[pl.ds(0, RPT * W)],
                    abits_hbm.at[pl.ds(wid * RPT * W, RPT * W)])


def _sc_gather(abits_hbm, x_hbm, tari_hbm, tarj_hbm,
               rowsi_hbm, rowsj_hbm, xi_hbm, xj_hbm,
               idx_v, rows_v, xrow_v, sem):
    wid = lax.axis_index("s") * NC + lax.axis_index("c")
    base = wid * BPW

    pltpu.sync_copy(tari_hbm.at[pl.ds(base, BPW)], idx_v)
    pltpu.async_copy(abits_hbm.at[idx_v], rows_v, sem).wait()
    pltpu.sync_copy(rows_v, rowsi_hbm.at[pl.ds(base, BPW)])
    pltpu.async_copy(x_hbm.at[idx_v], xrow_v, sem).wait()
    pltpu.sync_copy(xrow_v, xi_hbm.at[pl.ds(base, BPW)])

    pltpu.sync_copy(tarj_hbm.at[pl.ds(base, BPW)], idx_v)
    pltpu.async_copy(abits_hbm.at[idx_v], rows_v, sem).wait()
    pltpu.sync_copy(rows_v, rowsj_hbm.at[pl.ds(base, BPW)])
    pltpu.async_copy(x_hbm.at[idx_v], xrow_v, sem).wait()
    pltpu.sync_copy(xrow_v, xj_hbm.at[pl.ds(base, BPW)])


@functools.lru_cache(maxsize=None)
def _sc_kernels():
    mesh = plsc.VectorSubcoreMesh(core_axis_name="c", subcore_axis_name="s",
                                  num_cores=NC, num_subcores=NS)
    scan = pl.kernel(
        _sc_scan,
        out_type=jax.ShapeDtypeStruct((NW * NW * CAP,), jnp.int32),
        mesh=mesh,
        scratch_types=[
            pltpu.VMEM((NW * CAP,), jnp.int32),
            pltpu.VMEM((NW + L,), jnp.int32),
            pltpu.VMEM((EPW,), jnp.int32),
            pltpu.VMEM((EPW,), jnp.int32),
        ],
    )
    apply_ = pl.kernel(
        _sc_apply,
        out_type=jax.ShapeDtypeStruct((NROWS * W,), jnp.int32),
        mesh=mesh,
        scratch_types=[
            pltpu.VMEM((RPT * W + L,), jnp.int32),
            pltpu.VMEM((2 * WV * CAP,), jnp.int32),
            pltpu.SemaphoreType.DMA,
        ],
    )
    gather = pl.kernel(
        _sc_gather,
        out_type=(
            jax.ShapeDtypeStruct((B, W), jnp.int32),
            jax.ShapeDtypeStruct((B, W), jnp.int32),
            jax.ShapeDtypeStruct((B, D), jnp.float32),
            jax.ShapeDtypeStruct((B, D), jnp.float32),
        ),
        mesh=mesh,
        scratch_types=[
            pltpu.VMEM((BPW,), jnp.int32),
            pltpu.VMEM((BPW, W), jnp.int32),
            pltpu.VMEM((BPW, D), jnp.float32),
            pltpu.SemaphoreType.DMA,
        ],
    )
    return scan, apply_, gather


def _mlp(h, W1, b1, g, be, W2, b2):
    h = jnp.dot(h, W1, preferred_element_type=jnp.float32) + b1
    mu = jnp.mean(h, axis=-1, keepdims=True)
    var = jnp.mean((h - mu) ** 2, axis=-1, keepdims=True)
    h = (h - mu) * lax.rsqrt(var + EPS) * g + be
    h = jnp.maximum(h, 0.0)
    return jnp.dot(h, W2, preferred_element_type=jnp.float32) + b2


def _tc_body(rowsi_ref, rowsj_ref, xi_ref, xj_ref, xperm_ref,
             xcn_W1, xcn_b1, xcn_g, xcn_be, xcn_W2, xcn_b2,
             xij_W1, xij_b1, xij_g, xij_be, xij_W2, xij_b2,
             lin_W1, lin_b1, lin_g, lin_be, lin_W2, lin_b2,
             out_ref):
    cnw = rowsi_ref[...] & rowsj_ref[...]
    acc = jnp.zeros((BB, D), jnp.float32)
    for p in range(32):
        mask_p = jnp.int32(-(2 ** 31)) if p == 31 else jnp.int32(1 << p)
        bitf = (cnw & mask_p).astype(jnp.float32)
        acc = acc + jnp.dot(bitf, xperm_ref[p],
                            preferred_element_type=jnp.float32)
    xij = _mlp(xi_ref[...] * xj_ref[...],
               xij_W1[...], xij_b1[...], xij_g[...], xij_be[...],
               xij_W2[...], xij_b2[...])
    xcnh = _mlp(acc,
                xcn_W1[...], xcn_b1[...], xcn_g[...], xcn_be[...],
                xcn_W2[...], xcn_b2[...])
    out_ref[...] = _mlp(xcnh * BETA + xij,
                        lin_W1[...], lin_b1[...], lin_g[...], lin_be[...],
                        lin_W2[...], lin_b2[...])


def _tc_score(rowsi, rowsj, xi, xj, xperm, wts):
    full = lambda shape: pl.BlockSpec(shape, lambda i: (0,) * len(shape))
    in_specs = [
        pl.BlockSpec((BB, W), lambda i: (i, 0)),
        pl.BlockSpec((BB, W), lambda i: (i, 0)),
        pl.BlockSpec((BB, D), lambda i: (i, 0)),
        pl.BlockSpec((BB, D), lambda i: (i, 0)),
        full((32, W, D)),
    ] + [full(w.shape) for w in wts]
    return pl.pallas_call(
        _tc_body,
        grid=(B // BB,),
        in_specs=in_specs,
        out_specs=pl.BlockSpec((BB, OUT), lambda i: (i, 0)),
        out_shape=jax.ShapeDtypeStruct((B, OUT), jnp.float32),
        compiler_params=pltpu.CompilerParams(
            dimension_semantics=("arbitrary",)),
    )(rowsi, rowsj, xi, xj, xperm, *wts)


def kernel(x, edge_index, tar_ei,
           xcn_W1, xcn_b1, xcn_g, xcn_be, xcn_W2, xcn_b2,
           xij_W1, xij_b1, xij_g, xij_be, xij_W2, xij_b2,
           lin_W1, lin_b1, lin_g, lin_be, lin_W2, lin_b2):
    sc_scan, sc_apply, sc_gather = _sc_kernels()
    abits = sc_apply(sc_scan(edge_index[0], edge_index[1]))
    rowsi, rowsj, xi, xj = sc_gather(
        abits.reshape(NROWS, W), x, tar_ei[0], tar_ei[1])

    xpad = jnp.pad(x, ((0, NPAD - N), (0, 0)))
    scale = jnp.concatenate([2.0 ** -jnp.arange(31, dtype=jnp.float32),
                             jnp.array([-(2.0 ** -31)], jnp.float32)])
    xperm = xpad.reshape(W, 32, D).transpose(1, 0, 2) * scale[:, None, None]

    wts = [xcn_W1, xcn_b1.reshape(1, H), xcn_g.reshape(1, H),
           xcn_be.reshape(1, H), xcn_W2, xcn_b2.reshape(1, H),
           xij_W1, xij_b1.reshape(1, H), xij_g.reshape(1, H),
           xij_be.reshape(1, H), xij_W2, xij_b2.reshape(1, H),
           lin_W1, lin_b1.reshape(1, H), lin_g.reshape(1, H),
           lin_be.reshape(1, H), lin_W2, lin_b2.reshape(1, OUT)]
    return _tc_score(rowsi, rowsj, xi, xj, xperm, wts)


# speedup vs baseline: 6.2407x; 1.1183x over previous
import functools

import jax
import jax.numpy as jnp
from jax import lax
from jax.experimental import pallas as pl
from jax.experimental.pallas import tpu as pltpu
from jax.experimental.pallas import tpu_sc as plsc

N = 10000
E = 320000
B = 4096
D = 128
H = 128
OUT = 1
BETA = 1.0
EPS = 1e-5

L = 16
NC = 2
NS = 16
NW = NC * NS
W = 384
NPAD = 32 * W
RPT = 313
NROWS = NW * RPT
BPW = B // NW
BB = 256
EPW = E // NW
CAP = 512
MAGIC = 13401
SENT = RPT * W * 32
WV = 8

_GDN = lax.GatherDimensionNumbers(
    offset_dims=(), collapsed_slice_dims=(0,), start_index_map=(0,))


def _rot(a, idx):
    return lax.gather(a, idx[:, None], _GDN, (1,),
                      mode=lax.GatherScatterMode.PROMISE_IN_BOUNDS)


def _sc_scan(esrc_hbm, edst_hbm, bkt_hbm, cnt_hbm,
             bkt_v, cnt_v, dense_v, sbuf_v, dbuf_v):
    wid = lax.axis_index("s") * NC + lax.axis_index("c")
    lanes = lax.iota(jnp.int32, L)
    lane0 = lanes == 0
    zero16 = jnp.zeros((L,), jnp.int32)

    def cinit_body(i, carry):
        cnt_v[pl.ds(i * L, L)] = zero16
        return carry

    lax.fori_loop(0, NW, cinit_body, 0)

    e0 = wid * EPW
    pltpu.sync_copy(esrc_hbm.at[pl.ds(e0, EPW)], sbuf_v)
    pltpu.sync_copy(edst_hbm.at[pl.ds(e0, EPW)], dbuf_v)

    def scan_body(v, carry):
        s16 = sbuf_v[pl.ds(v * L, L)]
        d16 = dbuf_v[pl.ds(v * L, L)]
        ov = jnp.right_shift(s16 * MAGIC, 22)
        pv = s16 * (W * 32) + d16
        av = ov * CAP
        co = ov * L
        for l in range(L):
            rp = _rot(pv, (lanes + l) & (L - 1))
            cw = cnt_v[pl.ds(co[l], L)]
            bkt_v[pl.ds(av[l] + cw[0], L)] = rp
            cnt_v[pl.ds(co[l], L)] = jnp.minimum(cw + 1, CAP - L)
        return carry

    lax.fori_loop(0, EPW // L, scan_body, 0)

    for o in range(NW):
        cw = cnt_v[pl.ds(o * L, L)]
        bkt_v[pl.ds(o * CAP + cw[0], L)] = zero16 + (o + 1) * SENT
        dense_v[pl.ds(o, L)] = jnp.where(lane0, cw, dense_v[pl.ds(o, L)])

    pltpu.sync_copy(bkt_v, bkt_hbm.at[pl.ds(wid * NW * CAP, NW * CAP)])
    pltpu.sync_copy(dense_v.at[pl.ds(0, NW)],
                    cnt_hbm.at[pl.ds(wid * NW, NW)])


def _sc_apply(bkt_hbm, cnt_hbm, abits_hbm, bits_v, stg_v, cnt_v, sem):
    wid = lax.axis_index("s") * NC + lax.axis_index("c")
    lanes = lax.iota(jnp.int32, L)
    lane0 = lanes == 0
    zero16 = jnp.zeros((L,), jnp.int32)

    pltpu.sync_copy(cnt_hbm, cnt_v.at[pl.ds(0, NW * NW)])

    def issue(g):
        h = (g % 2) * (WV * CAP)
        return [pltpu.async_copy(
            bkt_hbm.at[pl.ds(((g * WV + k) * NW + wid) * CAP, CAP)],
            stg_v.at[pl.ds(h + k * CAP, CAP)], sem) for k in range(WV)]

    pend = {0: issue(0)}

    def zero_body(i, carry):
        bits_v[pl.ds(i * L, L)] = zero16
        return carry

    lax.fori_loop(0, (RPT * W + L) // L, zero_body, 0)
    pend[1] = issue(1)

    for g in range(NW // WV):
        for cp in pend.pop(g):
            cp.wait()
        h = (g % 2) * (WV * CAP)

        def bucket_body(k, carry, g=g, h=h):
            c = cnt_v[pl.ds((g * WV + k) * NW + wid, L)][0]
            base = h + k * CAP

            def vec_body(v, carry2):
                pv = stg_v[pl.ds(base + v * L, L)]
                wv = jnp.right_shift(pv, 5) - wid * (RPT * W)
                bv = jnp.left_shift(jnp.ones((L,), jnp.int32), pv & 31)
                for l in range(L):
                    wl = wv[l]
                    rb = _rot(bv, (lanes + l) & (L - 1))
                    old = bits_v[pl.ds(wl, L)]
                    bits_v[pl.ds(wl, L)] = jnp.where(lane0, old | rb, old)
                return carry2

            lax.fori_loop(0, jnp.right_shift(c + (L - 1), 4), vec_body, 0)
            return carry

        lax.fori_loop(0, WV, bucket_body, 0)
        if g + 2 < NW // WV:
            pend[g + 2] = issue(g + 2)

    pltpu.sync_copy(bits_v.at[pl.ds(0, RPT * W)],
                    abits_hbm.at[pl.ds(wid * RPT * W, RPT * W)])


def _sc_gather(abits_hbm, x_hbm, tari_hbm, tarj_hbm,
               rowsi_hbm, rowsj_hbm, xi_hbm, xj_hbm,
               idx_v, rows_v, xrow_v, sem):
    wid = lax.axis_index("s") * NC + lax.axis_index("c")
    base = wid * BPW

    pltpu.sync_copy(tari_hbm.at[pl.ds(base, BPW)], idx_v)
    pltpu.async_copy(abits_hbm.at[idx_v], rows_v, sem).wait()
    pltpu.sync_copy(rows_v, rowsi_hbm.at[pl.ds(base, BPW)])
    pltpu.async_copy(x_hbm.at[idx_v], xrow_v, sem).wait()
    pltpu.sync_copy(xrow_v, xi_hbm.at[pl.ds(base, BPW)])

    pltpu.sync_copy(tarj_hbm.at[pl.ds(base, BPW)], idx_v)
    pltpu.async_copy(abits_hbm.at[idx_v], rows_v, sem).wait()
    pltpu.sync_copy(rows_v, rowsj_hbm.at[pl.ds(base, BPW)])
    pltpu.async_copy(x_hbm.at[idx_v], xrow_v, sem).wait()
    pltpu.sync_copy(xrow_v, xj_hbm.at[pl.ds(base, BPW)])


@functools.lru_cache(maxsize=None)
def _sc_kernels():
    mesh = plsc.VectorSubcoreMesh(core_axis_name="c", subcore_axis_name="s",
                                  num_cores=NC, num_subcores=NS)
    scan = pl.kernel(
        _sc_scan,
        out_type=(
            jax.ShapeDtypeStruct((NW * NW * CAP,), jnp.int32),
            jax.ShapeDtypeStruct((NW * NW,), jnp.int32),
        ),
        mesh=mesh,
        scratch_types=[
            pltpu.VMEM((NW * CAP,), jnp.int32),
            pltpu.VMEM((NW * L,), jnp.int32),
            pltpu.VMEM((NW + L,), jnp.int32),
            pltpu.VMEM((EPW,), jnp.int32),
            pltpu.VMEM((EPW,), jnp.int32),
        ],
    )
    apply_ = pl.kernel(
        _sc_apply,
        out_type=jax.ShapeDtypeStruct((NROWS * W,), jnp.int32),
        mesh=mesh,
        scratch_types=[
            pltpu.VMEM((RPT * W + L,), jnp.int32),
            pltpu.VMEM((2 * WV * CAP,), jnp.int32),
            pltpu.VMEM((NW * NW + L,), jnp.int32),
            pltpu.SemaphoreType.DMA,
        ],
    )
    gather = pl.kernel(
        _sc_gather,
        out_type=(
            jax.ShapeDtypeStruct((B, W), jnp.int32),
            jax.ShapeDtypeStruct((B, W), jnp.int32),
            jax.ShapeDtypeStruct((B, D), jnp.float32),
            jax.ShapeDtypeStruct((B, D), jnp.float32),
        ),
        mesh=mesh,
        scratch_types=[
            pltpu.VMEM((BPW,), jnp.int32),
            pltpu.VMEM((BPW, W), jnp.int32),
            pltpu.VMEM((BPW, D), jnp.float32),
            pltpu.SemaphoreType.DMA,
        ],
    )
    return scan, apply_, gather


def _mlp(h, W1, b1, g, be, W2, b2):
    h = jnp.dot(h, W1, preferred_element_type=jnp.float32) + b1
    mu = jnp.mean(h, axis=-1, keepdims=True)
    var = jnp.mean((h - mu) ** 2, axis=-1, keepdims=True)
    h = (h - mu) * lax.rsqrt(var + EPS) * g + be
    h = jnp.maximum(h, 0.0)
    return jnp.dot(h, W2, preferred_element_type=jnp.float32) + b2


def _tc_body(rowsi_ref, rowsj_ref, xi_ref, xj_ref, xperm_ref,
             xcn_W1, xcn_b1, xcn_g, xcn_be, xcn_W2, xcn_b2,
             xij_W1, xij_b1, xij_g, xij_be, xij_W2, xij_b2,
             lin_W1, lin_b1, lin_g, lin_be, lin_W2, lin_b2,
             out_ref):
    cnw = rowsi_ref[...] & rowsj_ref[...]
    acc = jnp.zeros((BB, D), jnp.float32)
    for p in range(32):
        mask_p = jnp.int32(-(2 ** 31)) if p == 31 else jnp.int32(1 << p)
        bitf = (cnw & mask_p).astype(jnp.float32).astype(jnp.bfloat16)
        acc = acc + jnp.dot(bitf, xperm_ref[p],
                            preferred_element_type=jnp.float32)
    xij = _mlp(xi_ref[...] * xj_ref[...],
               xij_W1[...], xij_b1[...], xij_g[...], xij_be[...],
               xij_W2[...], xij_b2[...])
    xcnh = _mlp(acc,
                xcn_W1[...], xcn_b1[...], xcn_g[...], xcn_be[...],
                xcn_W2[...], xcn_b2[...])
    out_ref[...] = _mlp(xcnh * BETA + xij,
                        lin_W1[...], lin_b1[...], lin_g[...], lin_be[...],
                        lin_W2[...], lin_b2[...])


def _tc_score(rowsi, rowsj, xi, xj, xperm, wts):
    full = lambda shape: pl.BlockSpec(shape, lambda i: (0,) * len(shape))
    in_specs = [
        pl.BlockSpec((BB, W), lambda i: (i, 0)),
        pl.BlockSpec((BB, W), lambda i: (i, 0)),
        pl.BlockSpec((BB, D), lambda i: (i, 0)),
        pl.BlockSpec((BB, D), lambda i: (i, 0)),
        full((32, W, D)),
    ] + [full(w.shape) for w in wts]
    return pl.pallas_call(
        _tc_body,
        grid=(B // BB,),
        in_specs=in_specs,
        out_specs=pl.BlockSpec((BB, OUT), lambda i: (i, 0)),
        out_shape=jax.ShapeDtypeStruct((B, OUT), jnp.float32),
        compiler_params=pltpu.CompilerParams(
            dimension_semantics=("arbitrary",)),
    )(rowsi, rowsj, xi, xj, xperm, *wts)


def kernel(x, edge_index, tar_ei,
           xcn_W1, xcn_b1, xcn_g, xcn_be, xcn_W2, xcn_b2,
           xij_W1, xij_b1, xij_g, xij_be, xij_W2, xij_b2,
           lin_W1, lin_b1, lin_g, lin_be, lin_W2, lin_b2):
    sc_scan, sc_apply, sc_gather = _sc_kernels()
    abits = sc_apply(*sc_scan(edge_index[0], edge_index[1]))
    rowsi, rowsj, xi, xj = sc_gather(
        abits.reshape(NROWS, W), x, tar_ei[0], tar_ei[1])

    xpad = jnp.pad(x, ((0, NPAD - N), (0, 0)))
    scale = jnp.concatenate([2.0 ** -jnp.arange(31, dtype=jnp.float32),
                             jnp.array([-(2.0 ** -31)], jnp.float32)])
    xperm = xpad.reshape(W, 32, D).transpose(1, 0, 2) * scale[:, None, None]

    wts = [xcn_W1, xcn_b1.reshape(1, H), xcn_g.reshape(1, H),
           xcn_be.reshape(1, H), xcn_W2, xcn_b2.reshape(1, H),
           xij_W1, xij_b1.reshape(1, H), xij_g.reshape(1, H),
           xij_be.reshape(1, H), xij_W2, xij_b2.reshape(1, H),
           lin_W1, lin_b1.reshape(1, H), lin_g.reshape(1, H),
           lin_be.reshape(1, H), lin_W2, lin_b2.reshape(1, OUT)]
    return _tc_score(rowsi, rowsj, xi, xj, xperm, wts)
